# baseline (speedup 1.0000x reference)
import jax
import jax.numpy as jnp
from jax import lax
from jax.experimental import pallas as pl
from jax.experimental.pallas import tpu as pltpu
from jax.experimental.pallas import tpu_sc as plsc

NUM_EMBEDDINGS = 1000000
EMBEDDING_DIM = 64
BATCH = 16384
N_FIELDS = 26
PAD_F = 128
PAD_D = 128

NC, NS = 2, 16
NW = NC * NS
ROWS_PER_W = BATCH // NW
STAGE = 256
GCHUNK = 256
NHALF = ROWS_PER_W // GCHUNK
NUNIT = N_FIELDS * NHALF
NBUF = 2


def _gather_kernel(idx_hbm, table_hbm, out_hbm, idx_stage, idx_cols, rows_v, sems):
    wid = lax.axis_index("s") * NC + lax.axis_index("c")
    base = wid * ROWS_PER_W

    lane = lax.iota(jnp.int32, 16)
    for h in range(ROWS_PER_W // STAGE):
        pltpu.sync_copy(idx_hbm.at[pl.ds(base + h * STAGE, STAGE)], idx_stage)

        def shuffle_field(f, _, h=h):
            def shuffle_16(j, _):
                rows = j * 16 + lane
                cols = jnp.full((16,), 0, jnp.int32) + f
                v = plsc.load_gather(idx_stage, [rows, cols])
                idx_cols[f, pl.ds(h * STAGE + j * 16, 16)] = v << 1
                return ()

            lax.fori_loop(0, STAGE // 16, shuffle_16, ())
            return ()

        lax.fori_loop(0, N_FIELDS, shuffle_field, ())

    def start_unit(u, b):
        f = u // NHALF
        hh = u % NHALF
        pltpu.async_copy(
            table_hbm.at[idx_cols.at[f, pl.ds(hh * GCHUNK, GCHUNK)]],
            rows_v.at[b],
            sems.at[b],
        )

    def finish_unit(u, b):
        f = u // NHALF
        hh = u % NHALF
        pltpu.make_async_copy(
            table_hbm.at[idx_cols.at[f, pl.ds(hh * GCHUNK, GCHUNK)]],
            rows_v.at[b],
            sems.at[b],
        ).wait()
        pltpu.sync_copy(
            rows_v.at[b],
            out_hbm.at[pl.ds(base + hh * GCHUNK, GCHUNK), f],
        )

    for b in range(NBUF):
        start_unit(jnp.int32(b), b)

    def step(i, _):
        for b in range(NBUF):
            u = i + b
            finish_unit(u, b)

            @pl.when(u + NBUF < NUNIT)
            def _():
                start_unit(u + NBUF, b)
        return ()

    lax.fori_loop(0, NUNIT // NBUF, lambda i, c: step(i * NBUF, c), ())


@jax.jit
def _embedding_gather(idx_padded, weight):
    mesh = plsc.VectorSubcoreMesh(core_axis_name="c", subcore_axis_name="s")
    return pl.kernel(
        _gather_kernel,
        out_type=jax.ShapeDtypeStruct((BATCH, N_FIELDS, EMBEDDING_DIM), jnp.float32),
        mesh=mesh,
        scratch_types=[
            pltpu.VMEM((STAGE, PAD_F), jnp.int32),
            pltpu.VMEM((N_FIELDS, ROWS_PER_W), jnp.int32),
            pltpu.VMEM((NBUF, GCHUNK, EMBEDDING_DIM), jnp.float32),
            pltpu.SemaphoreType.DMA((NBUF,)),
        ],
        compiler_params=pltpu.CompilerParams(
            use_tc_tiling_on_sc=False, needs_layout_passes=False
        ),
    )(idx_padded, weight)


def kernel(input_, num_global_tokens, weight):
    del num_global_tokens
    idx_padded = jnp.pad(input_, ((0, 0), (0, PAD_F - N_FIELDS)))
    weight_padded = jnp.transpose(
        jnp.pad(weight.T, ((0, PAD_D - EMBEDDING_DIM), (0, 0)))
    )
    weight_view = jnp.reshape(weight_padded, (2 * NUM_EMBEDDINGS, EMBEDDING_DIM))
    return _embedding_gather(idx_padded, weight_view)


# speedup vs baseline: 1.0844x; 1.0844x over previous
import jax
import jax.numpy as jnp
from jax import lax
from jax.experimental import pallas as pl
from jax.experimental.pallas import tpu as pltpu
from jax.experimental.pallas import tpu_sc as plsc

NUM_EMBEDDINGS = 1000000
EMBEDDING_DIM = 64
BATCH = 16384
N_FIELDS = 26
PAD_F = 128
PAD_D = 128

NC, NS = 2, 16
NW = NC * NS
ROWS_PER_W = BATCH // NW
STAGE = 256
GCHUNK = 256
NHALF = ROWS_PER_W // GCHUNK
NUNIT = N_FIELDS * NHALF
NBUF = 2


def _gather_kernel(idx_hbm, table_hbm, out_hbm, idx_stage, idx_cols, rows_v, sems):
    wid = lax.axis_index("s") * NC + lax.axis_index("c")
    base = wid * ROWS_PER_W

    lane = lax.iota(jnp.int32, 16)
    for h in range(ROWS_PER_W // STAGE):
        pltpu.sync_copy(idx_hbm.at[pl.ds(base + h * STAGE, STAGE)], idx_stage)

        def shuffle_field(f, _, h=h):
            def shuffle_16(j, _):
                rows = j * 16 + lane
                cols = jnp.full((16,), 0, jnp.int32) + f
                v = plsc.load_gather(idx_stage, [rows, cols])
                idx_cols[f, pl.ds(h * STAGE + j * 16, 16)] = v << 1
                return ()

            lax.fori_loop(0, STAGE // 16, shuffle_16, ())
            return ()

        lax.fori_loop(0, N_FIELDS, shuffle_field, ())

    def start_unit(u, b):
        f = u // NHALF
        hh = u % NHALF
        pltpu.async_copy(
            table_hbm.at[idx_cols.at[f, pl.ds(hh * GCHUNK, GCHUNK)]],
            rows_v.at[b],
            sems.at[b],
        )

    def finish_unit(u, b):
        f = u // NHALF
        hh = u % NHALF
        pltpu.make_async_copy(
            table_hbm.at[idx_cols.at[f, pl.ds(hh * GCHUNK, GCHUNK)]],
            rows_v.at[b],
            sems.at[b],
        ).wait()
        pltpu.sync_copy(
            rows_v.at[b],
            out_hbm.at[pl.ds(base + hh * GCHUNK, GCHUNK), f],
        )

    for b in range(NBUF):
        start_unit(jnp.int32(b), b)

    def step(i, _):
        for b in range(NBUF):
            u = i + b
            finish_unit(u, b)

            @pl.when(u + NBUF < NUNIT)
            def _():
                start_unit(u + NBUF, b)
        return ()

    lax.fori_loop(0, NUNIT // NBUF, lambda i, c: step(i * NBUF, c), ())


@jax.jit
def _embedding_gather(idx_padded, weight):
    mesh = plsc.VectorSubcoreMesh(core_axis_name="c", subcore_axis_name="s")
    return pl.kernel(
        _gather_kernel,
        out_type=jax.ShapeDtypeStruct((BATCH, N_FIELDS, EMBEDDING_DIM), jnp.float32),
        mesh=mesh,
        scratch_types=[
            pltpu.VMEM((STAGE, PAD_F), jnp.int32),
            pltpu.VMEM((N_FIELDS, ROWS_PER_W), jnp.int32),
            pltpu.VMEM((NBUF, GCHUNK, EMBEDDING_DIM), jnp.float32),
            pltpu.SemaphoreType.DMA((NBUF,)),
        ],
        compiler_params=pltpu.CompilerParams(
            use_tc_tiling_on_sc=False, needs_layout_passes=False
        ),
    )(idx_padded, weight)


def kernel(input_, num_global_tokens, weight):
    del num_global_tokens
    idx_padded = jnp.pad(input_, ((0, 0), (0, PAD_F - N_FIELDS)))
    weight_padded = jnp.pad(weight, ((0, 0), (0, PAD_D - EMBEDDING_DIM)))
    weight_view = jnp.reshape(weight_padded, (2 * NUM_EMBEDDINGS, EMBEDDING_DIM))
    return _embedding_gather(idx_padded, weight_view)


# speedup vs baseline: 1.0923x; 1.0073x over previous
import jax
import jax.numpy as jnp
from jax import lax
from jax.experimental import pallas as pl
from jax.experimental.pallas import tpu as pltpu
from jax.experimental.pallas import tpu_sc as plsc

NUM_EMBEDDINGS = 1000000
EMBEDDING_DIM = 64
BATCH = 16384
N_FIELDS = 26
PAD_F = 128
PAD_D = 128

NC, NS = 2, 16
NW = NC * NS
ROWS_PER_W = BATCH // NW
STAGE = 256
GCHUNK = 256
NHALF = ROWS_PER_W // GCHUNK
NUNIT = N_FIELDS * NHALF
NBUF = 4


def _gather_kernel(idx_hbm, table_hbm, out_hbm, idx_stage, idx_cols, rows_v, sems):
    wid = lax.axis_index("s") * NC + lax.axis_index("c")
    base = wid * ROWS_PER_W

    lane = lax.iota(jnp.int32, 16)
    for h in range(ROWS_PER_W // STAGE):
        pltpu.sync_copy(idx_hbm.at[pl.ds(base + h * STAGE, STAGE)], idx_stage)

        def shuffle_field(f, _, h=h):
            def shuffle_16(j, _):
                rows = j * 16 + lane
                cols = jnp.full((16,), 0, jnp.int32) + f
                v = plsc.load_gather(idx_stage, [rows, cols])
                idx_cols[f, pl.ds(h * STAGE + j * 16, 16)] = v << 1
                return ()

            lax.fori_loop(0, STAGE // 16, shuffle_16, ())
            return ()

        lax.fori_loop(0, N_FIELDS, shuffle_field, ())

    def start_unit(u, b):
        f = u // NHALF
        hh = u % NHALF
        pltpu.async_copy(
            table_hbm.at[idx_cols.at[f, pl.ds(hh * GCHUNK, GCHUNK)]],
            rows_v.at[b],
            sems.at[b],
        )

    def finish_unit(u, b):
        f = u // NHALF
        hh = u % NHALF
        pltpu.make_async_copy(
            table_hbm.at[idx_cols.at[f, pl.ds(hh * GCHUNK, GCHUNK)]],
            rows_v.at[b],
            sems.at[b],
        ).wait()
        pltpu.sync_copy(
            rows_v.at[b],
            out_hbm.at[pl.ds(base + hh * GCHUNK, GCHUNK), f],
        )

    for b in range(NBUF):
        start_unit(jnp.int32(b), b)

    def step(i, _):
        for b in range(NBUF):
            u = i + b
            finish_unit(u, b)

            @pl.when(u + NBUF < NUNIT)
            def _():
                start_unit(u + NBUF, b)
        return ()

    lax.fori_loop(0, NUNIT // NBUF, lambda i, c: step(i * NBUF, c), ())


@jax.jit
def _embedding_gather(idx_padded, weight):
    mesh = plsc.VectorSubcoreMesh(core_axis_name="c", subcore_axis_name="s")
    return pl.kernel(
        _gather_kernel,
        out_type=jax.ShapeDtypeStruct((BATCH, N_FIELDS, EMBEDDING_DIM), jnp.float32),
        mesh=mesh,
        scratch_types=[
            pltpu.VMEM((STAGE, PAD_F), jnp.int32),
            pltpu.VMEM((N_FIELDS, ROWS_PER_W), jnp.int32),
            pltpu.VMEM((NBUF, GCHUNK, EMBEDDING_DIM), jnp.float32),
            pltpu.SemaphoreType.DMA((NBUF,)),
        ],
        compiler_params=pltpu.CompilerParams(
            use_tc_tiling_on_sc=False, needs_layout_passes=False
        ),
    )(idx_padded, weight)


def kernel(input_, num_global_tokens, weight):
    del num_global_tokens
    idx_padded = jnp.pad(input_, ((0, 0), (0, PAD_F - N_FIELDS)))
    weight_padded = jnp.pad(weight, ((0, 0), (0, PAD_D - EMBEDDING_DIM)))
    weight_view = jnp.reshape(weight_padded, (2 * NUM_EMBEDDINGS, EMBEDDING_DIM))
    return _embedding_gather(idx_padded, weight_view)
